# baseline (speedup 1.0000x reference)
import jax
import jax.numpy as jnp
from jax import lax
from jax.experimental import pallas as pl
from jax.experimental.pallas import tpu as pltpu
from jax.experimental.pallas import tpu_sc as plsc

N = 10000
NP = 10240
TRASH = 10000
E = 320000
K = 128
GS = 2
UNROLL = 4
EPT = 20480
E_PAD = EPT * 16
NCHUNK = EPT // K
NG = NCHUNK // GS
NC_ROWS = E_PAD // K
ZR = 128
ROWS_PER_TILE = NP // 16

_mesh = plsc.VectorSubcoreMesh(core_axis_name="c", subcore_axis_name="s")


def _make_edge_scatter(DH, with_deg):
    out_type = [jax.ShapeDtypeStruct((2, NP, DH), jnp.float32)]
    scratch = [
        pltpu.VMEM((4, GS, K), jnp.int32),
        pltpu.VMEM((4, GS, K), jnp.int32),
        pltpu.VMEM((2, GS, K, DH), jnp.float32),
        pltpu.VMEM_SHARED((NP, DH), jnp.float32),
        pltpu.VMEM_SHARED((NP, DH), jnp.float32),
        pltpu.SemaphoreType.DMA,
        pltpu.SemaphoreType.DMA,
        pltpu.SemaphoreType.DMA,
    ]
    if with_deg:
        out_type.append(jax.ShapeDtypeStruct((2, NP, 16), jnp.float32))
        scratch += [
            pltpu.VMEM((4, GS, K), jnp.int32),
            pltpu.VMEM((K, 16), jnp.float32),
            pltpu.VMEM_SHARED((NP, 16), jnp.float32),
            pltpu.SemaphoreType.DMA,
        ]

    def body(g_hbm, srcpr, dstsr, srcdr, s_out, deg_out, gixb, sixb,
             rowsb, gtab, acc, isem, gsem, ssem,
             dixb=None, ones_b=None, dacc=None, dsem=None):
        c = lax.axis_index("c")
        s = lax.axis_index("s")
        zeros16 = jnp.zeros((16,), jnp.float32)
        ones16 = jnp.ones((16,), jnp.float32)

        def _zrow(r, carry):
            for j in range(DH // 16):
                rowsb[0, 0, r, pl.ds(j * 16, 16)] = zeros16
            if with_deg:
                ones_b[r, pl.ds(0, 16)] = ones16
            return carry
        lax.fori_loop(0, ZR, _zrow, None)
        zbuf = rowsb.at[0, 0]

        for k in range(ROWS_PER_TILE // ZR):
            base = s * ROWS_PER_TILE + k * ZR
            pltpu.async_copy(g_hbm.at[c, pl.ds(base, ZR)],
                             gtab.at[pl.ds(base, ZR)], gsem)
            pltpu.sync_copy(zbuf, acc.at[pl.ds(base, ZR)])
            if with_deg:
                pltpu.sync_copy(rowsb.at[0, 0, :, pl.ds(0, 16)],
                                dacc.at[pl.ds(base, ZR)])
        for k in range(ROWS_PER_TILE // ZR):
            pltpu.make_async_copy(g_hbm.at[0, pl.ds(0, ZR)],
                                  gtab.at[pl.ds(0, ZR)], gsem).wait()
        plsc.subcore_barrier()

        row0 = s * NCHUNK

        def issue_idx(g, bank):
            r = row0 + g * GS
            pltpu.async_copy(srcpr.at[pl.ds(r, GS)], gixb.at[bank], isem)
            pltpu.async_copy(dstsr.at[pl.ds(r, GS)], sixb.at[bank], isem)
            if with_deg:
                pltpu.async_copy(srcdr.at[pl.ds(r, GS)], dixb.at[bank], isem)

        def drain_idx():
            n = 3 if with_deg else 2
            for _ in range(n):
                pltpu.make_async_copy(dstsr.at[pl.ds(0, GS)], gixb.at[0],
                                      isem).wait()

        def issue_gathers(bank3, bank2):
            for b in range(GS):
                pltpu.async_copy(gtab.at[gixb.at[bank3, b]],
                                 rowsb.at[bank2, b], gsem)

        def drain_rows(sem):
            for b in range(GS):
                pltpu.make_async_copy(gtab.at[pl.ds(0, K)],
                                      rowsb.at[0, b], sem).wait()

        def issue_scatters(g, bank3, bank2):
            for b in range(GS):
                pltpu.async_copy(rowsb.at[bank2, b],
                                 acc.at[sixb.at[bank3, b]], ssem, add=True)
            if with_deg:
                @pl.when((g * 2) // NG == c)
                def _():
                    for b in range(GS):
                        pltpu.async_copy(ones_b, dacc.at[dixb.at[bank3, b]],
                                         dsem, add=True)

        def drain_deg_cond(g):
            @pl.when((g * 2) // NG == c)
            def _():
                for b in range(GS):
                    pltpu.make_async_copy(deg_out.at[0, pl.ds(0, K)],
                                          ones_b, dsem).wait()

        issue_idx(0, 0)
        issue_idx(1, 1)
        drain_idx()
        issue_gathers(0, 0)

        def _group(t, carry):
            for u in range(UNROLL):
                j = t * UNROLL + u
                bank2 = u & 1
                bank3 = u
                @pl.when(j + 1 < NG)
                def _():
                    drain_idx()
                @pl.when(j >= 1)
                def _():
                    drain_rows(ssem)
                if with_deg:
                    @pl.when(j >= 1)
                    def _():
                        drain_deg_cond(j - 1)
                @pl.when(j + 1 < NG)
                def _():
                    issue_gathers((bank3 + 1) & 3, bank2 ^ 1)
                drain_rows(gsem)
                issue_scatters(j, bank3, bank2)
                @pl.when(j + 2 < NG)
                def _():
                    issue_idx(j + 2, (bank3 + 2) & 3)
            return carry
        lax.fori_loop(0, NG // UNROLL, _group, None)

        drain_rows(ssem)
        if with_deg:
            drain_deg_cond(NG - 1)
        plsc.subcore_barrier()

        for k in range(ROWS_PER_TILE // ZR):
            base = s * ROWS_PER_TILE + k * ZR
            pltpu.sync_copy(acc.at[pl.ds(base, ZR)],
                            s_out.at[c, pl.ds(base, ZR)])
            if with_deg:
                pltpu.sync_copy(dacc.at[pl.ds(base, ZR)],
                                deg_out.at[c, pl.ds(base, ZR)])

    if with_deg:
        def entry(g_hbm, srcpr, dstsr, srcdr, s_out, deg_out, gixb, sixb,
                  rowsb, gtab, acc, isem, gsem, ssem, dixb, ones_b,
                  dacc, dsem):
            body(g_hbm, srcpr, dstsr, srcdr, s_out, deg_out, gixb, sixb,
                 rowsb, gtab, acc, isem, gsem, ssem, dixb, ones_b,
                 dacc, dsem)
    else:
        def entry(g_hbm, srcpr, dstsr, s_out, gixb, sixb, rowsb, gtab,
                  acc, isem, gsem, ssem):
            body(g_hbm, srcpr, dstsr, None, s_out, None, gixb, sixb,
                 rowsb, gtab, acc, isem, gsem, ssem)

    return pl.kernel(entry, out_type=out_type, mesh=_mesh,
                     scratch_types=scratch,
                     compiler_params=pltpu.CompilerParams(
                         use_tc_tiling_on_sc=False))


_edge_scatter_deg_64 = _make_edge_scatter(64, True)
_edge_scatter_64 = _make_edge_scatter(64, False)
_edge_scatter_32 = _make_edge_scatter(32, False)


_R = 2048


def _mm_g_body(x_ref, w1_ref, g_ref):
    xb = x_ref[...]
    dh = w1_ref.shape[1] // 2
    g_ref[0] = jnp.dot(xb, w1_ref[:, pl.ds(0, dh)],
                       preferred_element_type=jnp.float32)
    g_ref[1] = jnp.dot(xb, w1_ref[:, pl.ds(dh, dh)],
                       preferred_element_type=jnp.float32)


def _mm_g(x, w1):
    n, d = x.shape
    dout = w1.shape[1]
    dh = dout // 2
    return pl.pallas_call(
        _mm_g_body,
        grid=(n // _R,),
        in_specs=[
            pl.BlockSpec((_R, d), lambda i: (i, 0)),
            pl.BlockSpec((d, dout), lambda i: (0, 0)),
        ],
        out_specs=pl.BlockSpec((2, _R, dh), lambda i: (0, i, 0)),
        out_shape=jax.ShapeDtypeStruct((2, n, dh), jnp.float32),
    )(x, w1)


def _mm_u_body(x_ref, w0_ref, u_ref):
    u_ref[...] = jnp.dot(x_ref[...], w0_ref[...],
                         preferred_element_type=jnp.float32)


def _mm_u(x, w0):
    n, d = x.shape
    dout = w0.shape[1]
    return pl.pallas_call(
        _mm_u_body,
        grid=(n // _R,),
        in_specs=[
            pl.BlockSpec((_R, d), lambda i: (i, 0)),
            pl.BlockSpec((d, dout), lambda i: (0, 0)),
        ],
        out_specs=pl.BlockSpec((_R, dout), lambda i: (i, 0)),
        out_shape=jax.ShapeDtypeStruct((n, dout), jnp.float32),
    )(x, w0)


def _comb_g_body(u_ref, g_ref, sp_ref, dp_ref, b_ref, w1_ref, h_ref, g2_ref):
    g = jnp.concatenate([g_ref[0], g_ref[1]], axis=-1)
    sadd = jnp.concatenate([sp_ref[0], sp_ref[1]], axis=-1)
    dsum = dp_ref[0] + dp_ref[1] - 1.0
    h = u_ref[...] + dsum * g - sadd + b_ref[...]
    h = jnp.maximum(h, 0.0)
    h_ref[...] = h
    dh = w1_ref.shape[1] // 2
    g2_ref[0] = jnp.dot(h, w1_ref[:, pl.ds(0, dh)],
                        preferred_element_type=jnp.float32)
    g2_ref[1] = jnp.dot(h, w1_ref[:, pl.ds(dh, dh)],
                        preferred_element_type=jnp.float32)


def _comb_g(u, g, sp, dp, b, w1):
    n, d = u.shape
    dhin = d // 2
    dout = w1.shape[1]
    dh = dout // 2
    return pl.pallas_call(
        _comb_g_body,
        grid=(n // _R,),
        in_specs=[
            pl.BlockSpec((_R, d), lambda i: (i, 0)),
            pl.BlockSpec((2, _R, dhin), lambda i: (0, i, 0)),
            pl.BlockSpec((2, _R, dhin), lambda i: (0, i, 0)),
            pl.BlockSpec((2, _R, 1), lambda i: (0, i, 0)),
            pl.BlockSpec((1, d), lambda i: (0, 0)),
            pl.BlockSpec((d, dout), lambda i: (0, 0)),
        ],
        out_specs=[
            pl.BlockSpec((_R, d), lambda i: (i, 0)),
            pl.BlockSpec((2, _R, dh), lambda i: (0, i, 0)),
        ],
        out_shape=[
            jax.ShapeDtypeStruct((n, d), jnp.float32),
            jax.ShapeDtypeStruct((2, n, dh), jnp.float32),
        ],
    )(u, g, sp, dp, b, w1)


def _final_body(u_ref, g_ref, sp_ref, dp_ref, b_ref, o_ref):
    g = jnp.concatenate([g_ref[0], g_ref[1]], axis=-1)
    sadd = jnp.concatenate([sp_ref[0], sp_ref[1]], axis=-1)
    dsum = dp_ref[0] + dp_ref[1] - 1.0
    o = u_ref[...] + dsum * g - sadd + b_ref[...]
    col = lax.broadcasted_iota(jnp.int32, o.shape, 1)
    o = jnp.where(col < 40, o, jnp.float32(-1e30))
    m = jnp.max(o, axis=1, keepdims=True)
    ex = jnp.exp(o - m)
    lse = jnp.log(jnp.sum(ex, axis=1, keepdims=True)) + m
    o_ref[...] = o - lse


def _final(u, g, sp, dp, b):
    n, d = u.shape
    dhin = d // 2
    return pl.pallas_call(
        _final_body,
        grid=(n // _R,),
        in_specs=[
            pl.BlockSpec((_R, d), lambda i: (i, 0)),
            pl.BlockSpec((2, _R, dhin), lambda i: (0, i, 0)),
            pl.BlockSpec((2, _R, dhin), lambda i: (0, i, 0)),
            pl.BlockSpec((2, _R, 1), lambda i: (0, i, 0)),
            pl.BlockSpec((1, d), lambda i: (0, 0)),
        ],
        out_specs=pl.BlockSpec((_R, d), lambda i: (i, 0)),
        out_shape=jax.ShapeDtypeStruct((n, d), jnp.float32),
    )(u, g, sp, dp, b)


def kernel(x, adj_t, W0_1, W1_1, b1, W0_2, W1_2, b2, W0_3, W1_3, b3):
    src = adj_t[0].astype(jnp.int32)
    dst = adj_t[1].astype(jnp.int32)
    self_loop = src == dst
    pad = E_PAD - E
    srcpr = jnp.concatenate([src, jnp.zeros((pad,), jnp.int32)]
                            ).reshape(NC_ROWS, K)
    dsts = jnp.concatenate([jnp.where(self_loop, TRASH, dst),
                            jnp.full((pad,), TRASH, jnp.int32)]
                           ).reshape(NC_ROWS, K)
    srcd = jnp.concatenate([jnp.where(self_loop, TRASH, src),
                            jnp.full((pad,), TRASH, jnp.int32)]
                           ).reshape(NC_ROWS, K)

    xp = jnp.pad(x, ((0, NP - N), (0, 0)))
    w0_3p = jnp.pad(W0_3, ((0, 0), (0, 24)))
    w1_3p = jnp.pad(W1_3, ((0, 0), (0, 24)))
    b1r = b1.reshape(1, -1)
    b2r = b2.reshape(1, -1)
    b3r = jnp.pad(b3, (0, 24)).reshape(1, -1)

    g1 = _mm_g(xp, W1_1)
    s1p, degp = _edge_scatter_deg_64(g1, srcpr, dsts, srcd)
    u1 = _mm_u(xp, W0_1)
    dp = degp[:, :, 0:1]
    h2, g2 = _comb_g(u1, g1, s1p, dp, b1r, W1_2)
    (s2p,) = _edge_scatter_64(g2, srcpr, dsts)
    u2 = _mm_u(h2, W0_2)
    h3, g3 = _comb_g(u2, g2, s2p, dp, b2r, w1_3p)
    (s3p,) = _edge_scatter_32(g3, srcpr, dsts)
    u3 = _mm_u(h3, w0_3p)
    o = _final(u3, g3, s3p, dp, b3r)
    return o[:N, :40]


# speedup vs baseline: 1.0068x; 1.0068x over previous
import jax
import jax.numpy as jnp
from jax import lax
from jax.experimental import pallas as pl
from jax.experimental.pallas import tpu as pltpu
from jax.experimental.pallas import tpu_sc as plsc

N = 10000
NP = 10240
TRASH = 10000
E = 320000
K = 128
GS = 2
UNROLL = 4
EPT = 20480
E_PAD = EPT * 16
NCHUNK = EPT // K
NG = NCHUNK // GS
NC_ROWS = E_PAD // K
ZR = 128
ROWS_PER_TILE = NP // 16

_mesh = plsc.VectorSubcoreMesh(core_axis_name="c", subcore_axis_name="s")


def _make_edge_scatter(DH, with_deg):
    out_type = [jax.ShapeDtypeStruct((2, NP, DH), jnp.float32)]
    scratch = [
        pltpu.VMEM((4, GS, K), jnp.int32),
        pltpu.VMEM((4, GS, K), jnp.int32),
        pltpu.VMEM((2, GS, K, DH), jnp.float32),
        pltpu.VMEM_SHARED((NP, DH), jnp.float32),
        pltpu.VMEM_SHARED((NP, DH), jnp.float32),
        pltpu.SemaphoreType.DMA,
        pltpu.SemaphoreType.DMA,
        pltpu.SemaphoreType.DMA,
    ]
    if with_deg:
        out_type.append(jax.ShapeDtypeStruct((2, NP, 16), jnp.float32))
        scratch += [
            pltpu.VMEM((4, GS, K), jnp.int32),
            pltpu.VMEM((K, 16), jnp.float32),
            pltpu.VMEM_SHARED((NP, 16), jnp.float32),
            pltpu.SemaphoreType.DMA,
        ]

    def body(g_hbm, srcpr, dstsr, srcdr, s_out, deg_out, gixb, sixb,
             rowsb, gtab, acc, isem, gsem, ssem,
             dixb=None, ones_b=None, dacc=None, dsem=None):
        c = lax.axis_index("c")
        s = lax.axis_index("s")
        zeros16 = jnp.zeros((16,), jnp.float32)
        ones16 = jnp.ones((16,), jnp.float32)
        row0 = s * NCHUNK

        def issue_idx(g, bank):
            r = row0 + g * GS
            pltpu.async_copy(srcpr.at[pl.ds(r, GS)], gixb.at[bank], isem)
            pltpu.async_copy(dstsr.at[pl.ds(r, GS)], sixb.at[bank], isem)
            if with_deg:
                pltpu.async_copy(srcdr.at[pl.ds(r, GS)], dixb.at[bank], isem)

        issue_idx(0, 0)
        issue_idx(1, 1)

        def _zrow(r, carry):
            for j in range(DH // 16):
                rowsb[0, 0, r, pl.ds(j * 16, 16)] = zeros16
            if with_deg:
                ones_b[r, pl.ds(0, 16)] = ones16
            return carry
        lax.fori_loop(0, ZR, _zrow, None)
        zbuf = rowsb.at[0, 0]

        for k in range(ROWS_PER_TILE // ZR):
            base = s * ROWS_PER_TILE + k * ZR
            pltpu.async_copy(g_hbm.at[c, pl.ds(base, ZR)],
                             gtab.at[pl.ds(base, ZR)], gsem)
            pltpu.sync_copy(zbuf, acc.at[pl.ds(base, ZR)])
            if with_deg:
                pltpu.sync_copy(rowsb.at[0, 0, :, pl.ds(0, 16)],
                                dacc.at[pl.ds(base, ZR)])
        for k in range(ROWS_PER_TILE // ZR):
            pltpu.make_async_copy(g_hbm.at[0, pl.ds(0, ZR)],
                                  gtab.at[pl.ds(0, ZR)], gsem).wait()
        plsc.subcore_barrier()

        def drain_idx():
            n = 3 if with_deg else 2
            for _ in range(n):
                pltpu.make_async_copy(dstsr.at[pl.ds(0, GS)], gixb.at[0],
                                      isem).wait()

        def issue_gathers(bank3, bank2):
            for b in range(GS):
                pltpu.async_copy(gtab.at[gixb.at[bank3, b]],
                                 rowsb.at[bank2, b], gsem)

        def drain_rows(sem):
            for b in range(GS):
                pltpu.make_async_copy(gtab.at[pl.ds(0, K)],
                                      rowsb.at[0, b], sem).wait()

        def issue_scatters(g, bank3, bank2):
            for b in range(GS):
                pltpu.async_copy(rowsb.at[bank2, b],
                                 acc.at[sixb.at[bank3, b]], ssem, add=True)
            if with_deg:
                @pl.when((g * 2) // NG == c)
                def _():
                    for b in range(GS):
                        pltpu.async_copy(ones_b, dacc.at[dixb.at[bank3, b]],
                                         dsem, add=True)

        def drain_deg_cond(g):
            @pl.when((g * 2) // NG == c)
            def _():
                for b in range(GS):
                    pltpu.make_async_copy(deg_out.at[0, pl.ds(0, K)],
                                          ones_b, dsem).wait()

        drain_idx()
        issue_gathers(0, 0)

        def _group(t, carry):
            for u in range(UNROLL):
                j = t * UNROLL + u
                bank2 = u & 1
                bank3 = u
                @pl.when(j + 1 < NG)
                def _():
                    drain_idx()
                @pl.when(j >= 1)
                def _():
                    drain_rows(ssem)
                if with_deg:
                    @pl.when(j >= 1)
                    def _():
                        drain_deg_cond(j - 1)
                @pl.when(j + 1 < NG)
                def _():
                    issue_gathers((bank3 + 1) & 3, bank2 ^ 1)
                drain_rows(gsem)
                issue_scatters(j, bank3, bank2)
                @pl.when(j + 2 < NG)
                def _():
                    issue_idx(j + 2, (bank3 + 2) & 3)
            return carry
        lax.fori_loop(0, NG // UNROLL, _group, None)

        drain_rows(ssem)
        if with_deg:
            drain_deg_cond(NG - 1)
        plsc.subcore_barrier()

        for k in range(ROWS_PER_TILE // ZR):
            base = s * ROWS_PER_TILE + k * ZR
            pltpu.sync_copy(acc.at[pl.ds(base, ZR)],
                            s_out.at[c, pl.ds(base, ZR)])
            if with_deg:
                pltpu.sync_copy(dacc.at[pl.ds(base, ZR)],
                                deg_out.at[c, pl.ds(base, ZR)])

    if with_deg:
        def entry(g_hbm, srcpr, dstsr, srcdr, s_out, deg_out, gixb, sixb,
                  rowsb, gtab, acc, isem, gsem, ssem, dixb, ones_b,
                  dacc, dsem):
            body(g_hbm, srcpr, dstsr, srcdr, s_out, deg_out, gixb, sixb,
                 rowsb, gtab, acc, isem, gsem, ssem, dixb, ones_b,
                 dacc, dsem)
    else:
        def entry(g_hbm, srcpr, dstsr, s_out, gixb, sixb, rowsb, gtab,
                  acc, isem, gsem, ssem):
            body(g_hbm, srcpr, dstsr, None, s_out, None, gixb, sixb,
                 rowsb, gtab, acc, isem, gsem, ssem)

    return pl.kernel(entry, out_type=out_type, mesh=_mesh,
                     scratch_types=scratch,
                     compiler_params=pltpu.CompilerParams(
                         use_tc_tiling_on_sc=False))


_edge_scatter_deg_64 = _make_edge_scatter(64, True)
_edge_scatter_64 = _make_edge_scatter(64, False)
_edge_scatter_32 = _make_edge_scatter(32, False)


_R = 2048


def _mm2_body(x_ref, w0_ref, w1_ref, u_ref, g_ref):
    xb = x_ref[...]
    dh = w0_ref.shape[1] // 2
    u_ref[...] = jnp.dot(xb, w0_ref[...], preferred_element_type=jnp.float32)
    g_ref[0] = jnp.dot(xb, w1_ref[:, pl.ds(0, dh)],
                       preferred_element_type=jnp.float32)
    g_ref[1] = jnp.dot(xb, w1_ref[:, pl.ds(dh, dh)],
                       preferred_element_type=jnp.float32)


def _mm2(x, w0, w1):
    n, d = x.shape
    dout = w0.shape[1]
    dh = dout // 2
    u, g = pl.pallas_call(
        _mm2_body,
        grid=(n // _R,),
        in_specs=[
            pl.BlockSpec((_R, d), lambda i: (i, 0)),
            pl.BlockSpec((d, dout), lambda i: (0, 0)),
            pl.BlockSpec((d, dout), lambda i: (0, 0)),
        ],
        out_specs=[
            pl.BlockSpec((_R, dout), lambda i: (i, 0)),
            pl.BlockSpec((2, _R, dh), lambda i: (0, i, 0)),
        ],
        out_shape=[
            jax.ShapeDtypeStruct((n, dout), jnp.float32),
            jax.ShapeDtypeStruct((2, n, dh), jnp.float32),
        ],
    )(x, w0, w1)
    return u, g


def _comb_mm2_body(u_ref, g_ref, sp_ref, dp_ref, b_ref, w0_ref, w1_ref,
                   u2_ref, g2_ref):
    g = jnp.concatenate([g_ref[0], g_ref[1]], axis=-1)
    sadd = jnp.concatenate([sp_ref[0], sp_ref[1]], axis=-1)
    dsum = dp_ref[0] + dp_ref[1] - 1.0
    h = u_ref[...] + dsum * g - sadd + b_ref[...]
    h = jnp.maximum(h, 0.0)
    dh = w0_ref.shape[1] // 2
    u2_ref[...] = jnp.dot(h, w0_ref[...], preferred_element_type=jnp.float32)
    g2_ref[0] = jnp.dot(h, w1_ref[:, pl.ds(0, dh)],
                        preferred_element_type=jnp.float32)
    g2_ref[1] = jnp.dot(h, w1_ref[:, pl.ds(dh, dh)],
                        preferred_element_type=jnp.float32)


def _comb_mm2(u, g, sp, dp, b, w0, w1):
    n, d = u.shape
    dhin = d // 2
    dout = w0.shape[1]
    dh = dout // 2
    return pl.pallas_call(
        _comb_mm2_body,
        grid=(n // _R,),
        in_specs=[
            pl.BlockSpec((_R, d), lambda i: (i, 0)),
            pl.BlockSpec((2, _R, dhin), lambda i: (0, i, 0)),
            pl.BlockSpec((2, _R, dhin), lambda i: (0, i, 0)),
            pl.BlockSpec((2, _R, 1), lambda i: (0, i, 0)),
            pl.BlockSpec((1, d), lambda i: (0, 0)),
            pl.BlockSpec((d, dout), lambda i: (0, 0)),
            pl.BlockSpec((d, dout), lambda i: (0, 0)),
        ],
        out_specs=[
            pl.BlockSpec((_R, dout), lambda i: (i, 0)),
            pl.BlockSpec((2, _R, dh), lambda i: (0, i, 0)),
        ],
        out_shape=[
            jax.ShapeDtypeStruct((n, dout), jnp.float32),
            jax.ShapeDtypeStruct((2, n, dh), jnp.float32),
        ],
    )(u, g, sp, dp, b, w0, w1)


def _final_body(u_ref, g_ref, sp_ref, dp_ref, b_ref, o_ref):
    g = jnp.concatenate([g_ref[0], g_ref[1]], axis=-1)
    sadd = jnp.concatenate([sp_ref[0], sp_ref[1]], axis=-1)
    dsum = dp_ref[0] + dp_ref[1] - 1.0
    o = u_ref[...] + dsum * g - sadd + b_ref[...]
    col = lax.broadcasted_iota(jnp.int32, o.shape, 1)
    o = jnp.where(col < 40, o, jnp.float32(-1e30))
    m = jnp.max(o, axis=1, keepdims=True)
    ex = jnp.exp(o - m)
    lse = jnp.log(jnp.sum(ex, axis=1, keepdims=True)) + m
    o_ref[...] = o - lse


def _final(u, g, sp, dp, b):
    n, d = u.shape
    dhin = d // 2
    return pl.pallas_call(
        _final_body,
        grid=(n // _R,),
        in_specs=[
            pl.BlockSpec((_R, d), lambda i: (i, 0)),
            pl.BlockSpec((2, _R, dhin), lambda i: (0, i, 0)),
            pl.BlockSpec((2, _R, dhin), lambda i: (0, i, 0)),
            pl.BlockSpec((2, _R, 1), lambda i: (0, i, 0)),
            pl.BlockSpec((1, d), lambda i: (0, 0)),
        ],
        out_specs=pl.BlockSpec((_R, d), lambda i: (i, 0)),
        out_shape=jax.ShapeDtypeStruct((n, d), jnp.float32),
    )(u, g, sp, dp, b)


def kernel(x, adj_t, W0_1, W1_1, b1, W0_2, W1_2, b2, W0_3, W1_3, b3):
    src = adj_t[0].astype(jnp.int32)
    dst = adj_t[1].astype(jnp.int32)
    self_loop = src == dst
    pad = E_PAD - E
    srcpr = jnp.concatenate([src, jnp.zeros((pad,), jnp.int32)]
                            ).reshape(NC_ROWS, K)
    dsts = jnp.concatenate([jnp.where(self_loop, TRASH, dst),
                            jnp.full((pad,), TRASH, jnp.int32)]
                           ).reshape(NC_ROWS, K)
    srcd = jnp.concatenate([jnp.where(self_loop, TRASH, src),
                            jnp.full((pad,), TRASH, jnp.int32)]
                           ).reshape(NC_ROWS, K)

    xp = jnp.pad(x, ((0, NP - N), (0, 0)))
    w0_3p = jnp.pad(W0_3, ((0, 0), (0, 24)))
    w1_3p = jnp.pad(W1_3, ((0, 0), (0, 24)))
    b1r = b1.reshape(1, -1)
    b2r = b2.reshape(1, -1)
    b3r = jnp.pad(b3, (0, 24)).reshape(1, -1)

    u1, g1 = _mm2(xp, W0_1, W1_1)
    s1p, degp = _edge_scatter_deg_64(g1, srcpr, dsts, srcd)
    dp = degp[:, :, 0:1]
    u2, g2 = _comb_mm2(u1, g1, s1p, dp, b1r, W0_2, W1_2)
    (s2p,) = _edge_scatter_64(g2, srcpr, dsts)
    u3, g3 = _comb_mm2(u2, g2, s2p, dp, b2r, w0_3p, w1_3p)
    (s3p,) = _edge_scatter_32(g3, srcpr, dsts)
    o = _final(u3, g3, s3p, dp, b3r)
    return o[:N, :40]
